# baseline (speedup 1.0000x reference)
import functools

import jax
import jax.numpy as jnp
from jax import lax
from jax.experimental import pallas as pl
from jax.experimental.pallas import tpu as pltpu
from jax.experimental.pallas import tpu_sc as plsc

_VOCAB = 1000000
_DIM = 64
_SEQ = 200
_BATCH = 4096
_B = _SEQ * _BATCH

_NC, _NS = 2, 16
_NW = _NC * _NS
_PER_W = _B // _NW
_SUB = 128
_K = 8
_CHUNK = _K * _SUB
_NCHUNK = _PER_W // _CHUNK

_mesh = plsc.VectorSubcoreMesh(core_axis_name="c", subcore_axis_name="s")


@functools.partial(
    pl.kernel,
    mesh=_mesh,
    out_type=jax.ShapeDtypeStruct((_B, _DIM), jnp.float32),
    scratch_types=[
        pltpu.VMEM((_K, _SUB), jnp.int32),
        pltpu.VMEM((_CHUNK, _DIM), jnp.float32),
        pltpu.SemaphoreType.DMA,
    ],
    compiler_params=pltpu.CompilerParams(use_tc_tiling_on_sc=False),
)
def _emb_lookup(idx_hbm, table_hbm, out_hbm, idx_v, rows_v, sem):
    wid = lax.axis_index("s") * _NC + lax.axis_index("c")
    base = wid * _PER_W

    def body(c, carry):
        off = pl.multiple_of(base + c * _CHUNK, _CHUNK)
        pltpu.sync_copy(idx_hbm.at[pl.ds(pl.multiple_of(off // _SUB, _K), _K)], idx_v)
        copies = [
            pltpu.async_copy(
                table_hbm.at[idx_v.at[j]],
                rows_v.at[pl.ds(j * _SUB, _SUB)],
                sem,
            )
            for j in range(_K)
        ]
        for cp in copies:
            cp.wait()
        pltpu.sync_copy(rows_v, out_hbm.at[pl.ds(off, _CHUNK)])
        return carry

    lax.fori_loop(0, _NCHUNK, body, 0)


def kernel(src_input, word_lut):
    idx = src_input.reshape(_B // _SUB, _SUB)
    out = _emb_lookup(idx, word_lut)
    return out.reshape(_SEQ, _BATCH, _DIM)


# speedup vs baseline: 1.0156x; 1.0156x over previous
import functools

import jax
import jax.numpy as jnp
from jax import lax
from jax.experimental import pallas as pl
from jax.experimental.pallas import tpu as pltpu
from jax.experimental.pallas import tpu_sc as plsc

_VOCAB = 1000000
_DIM = 64
_SEQ = 200
_BATCH = 4096
_B = _SEQ * _BATCH

_NC, _NS = 2, 16
_NW = _NC * _NS
_PER_W = _B // _NW
_SUB = 128
_NROW = _PER_W // _SUB
_K = 5
_CHUNK = _K * _SUB
_NCHUNK = _PER_W // _CHUNK
_NPAIR = _NCHUNK // 2

_mesh = plsc.VectorSubcoreMesh(core_axis_name="c", subcore_axis_name="s")


@functools.partial(
    pl.kernel,
    mesh=_mesh,
    out_type=jax.ShapeDtypeStruct((_B, _DIM), jnp.float32),
    scratch_types=[
        pltpu.VMEM((_NROW, _SUB), jnp.int32),
        pltpu.VMEM((_CHUNK, _DIM), jnp.float32),
        pltpu.VMEM((_CHUNK, _DIM), jnp.float32),
        pltpu.SemaphoreType.DMA,
        pltpu.SemaphoreType.DMA,
        pltpu.SemaphoreType.DMA,
        pltpu.SemaphoreType.DMA,
    ],
    compiler_params=pltpu.CompilerParams(use_tc_tiling_on_sc=False),
)
def _emb_lookup(idx_hbm, table_hbm, out_hbm, idx_v, rows0, rows1,
                gsem0, gsem1, osem0, osem1):
    wid = lax.axis_index("s") * _NC + lax.axis_index("c")
    base = pl.multiple_of(wid * _PER_W, _PER_W)
    rows = (rows0, rows1)
    gsem = (gsem0, gsem1)
    osem = (osem0, osem1)

    pltpu.sync_copy(idx_hbm.at[pl.ds(pl.multiple_of(base // _SUB, _NROW), _NROW)],
                    idx_v)

    def fire(c, b):
        for j in range(_K):
            pltpu.async_copy(
                table_hbm.at[idx_v.at[c * _K + j]],
                rows[b].at[pl.ds(j * _SUB, _SUB)],
                gsem[b],
            )

    def drain_gather(b):
        pltpu.make_async_copy(table_hbm.at[pl.ds(0, _CHUNK)], rows[b],
                              gsem[b]).wait()

    def out_start(c, b):
        pltpu.async_copy(
            rows[b],
            out_hbm.at[pl.ds(pl.multiple_of(base + c * _CHUNK, _CHUNK), _CHUNK)],
            osem[b],
        )

    def drain_out(b):
        pltpu.make_async_copy(rows[b], out_hbm.at[pl.ds(0, _CHUNK)],
                              osem[b]).wait()

    fire(0, 0)
    drain_gather(0)
    out_start(0, 0)
    fire(1, 1)
    drain_gather(1)
    out_start(1, 1)
    drain_out(0)
    fire(2, 0)

    def body(t, carry):
        c0 = 2 * t
        drain_gather(0)
        out_start(c0, 0)
        drain_out(1)
        fire(c0 + 1, 1)
        drain_gather(1)
        out_start(c0 + 1, 1)
        drain_out(0)
        fire(c0 + 2, 0)
        return carry

    lax.fori_loop(1, _NPAIR - 1, body, 0)

    drain_gather(0)
    out_start(_NCHUNK - 2, 0)
    drain_out(1)
    fire(_NCHUNK - 1, 1)
    drain_gather(1)
    out_start(_NCHUNK - 1, 1)
    drain_out(0)
    drain_out(1)


def kernel(src_input, word_lut):
    idx = src_input.reshape(_B // _SUB, _SUB)
    out = _emb_lookup(idx, word_lut)
    return out.reshape(_SEQ, _BATCH, _DIM)


# speedup vs baseline: 1.0177x; 1.0020x over previous
import functools

import jax
import jax.numpy as jnp
from jax import lax
from jax.experimental import pallas as pl
from jax.experimental.pallas import tpu as pltpu
from jax.experimental.pallas import tpu_sc as plsc

_VOCAB = 1000000
_DIM = 64
_SEQ = 200
_BATCH = 4096

_NC, _NS = 2, 16
_NW = _NC * _NS
_COLS = _BATCH // _NW
_G = 5
_NCHUNK = _SEQ // _G
_NPAIR = _NCHUNK // 2

_mesh = plsc.VectorSubcoreMesh(core_axis_name="c", subcore_axis_name="s")


@functools.partial(
    pl.kernel,
    mesh=_mesh,
    out_type=jax.ShapeDtypeStruct((_SEQ, _BATCH, _DIM), jnp.float32),
    scratch_types=[
        pltpu.VMEM((_SEQ, _COLS), jnp.int32),
        pltpu.VMEM((_G, _COLS, _DIM), jnp.float32),
        pltpu.VMEM((_G, _COLS, _DIM), jnp.float32),
        pltpu.SemaphoreType.DMA,
        pltpu.SemaphoreType.DMA,
        pltpu.SemaphoreType.DMA,
        pltpu.SemaphoreType.DMA,
    ],
    compiler_params=pltpu.CompilerParams(use_tc_tiling_on_sc=False),
)
def _emb_lookup(idx_hbm, table_hbm, out_hbm, idx_v, rows0, rows1,
                gsem0, gsem1, osem0, osem1):
    wid = lax.axis_index("s") * _NC + lax.axis_index("c")
    col = pl.multiple_of(wid * _COLS, _COLS)
    rows = (rows0, rows1)
    gsem = (gsem0, gsem1)
    osem = (osem0, osem1)

    pltpu.sync_copy(idx_hbm.at[pl.ds(0, _SEQ), pl.ds(col, _COLS)], idx_v)

    def fire(c, b):
        for g in range(_G):
            pltpu.async_copy(
                table_hbm.at[idx_v.at[c * _G + g]],
                rows[b].at[g],
                gsem[b],
            )

    def drain_gather(b):
        pltpu.make_async_copy(
            out_hbm.at[pl.ds(0, _G), pl.ds(0, _COLS)], rows[b], gsem[b]
        ).wait()

    def out_start(c, b):
        pltpu.async_copy(
            rows[b],
            out_hbm.at[pl.ds(c * _G, _G), pl.ds(col, _COLS)],
            osem[b],
        )

    def drain_out(b):
        pltpu.make_async_copy(
            rows[b], out_hbm.at[pl.ds(0, _G), pl.ds(0, _COLS)], osem[b]
        ).wait()

    fire(0, 0)
    drain_gather(0)
    out_start(0, 0)
    fire(1, 1)
    drain_gather(1)
    out_start(1, 1)
    drain_out(0)
    fire(2, 0)

    def body(t, carry):
        c0 = 2 * t
        drain_gather(0)
        out_start(c0, 0)
        drain_out(1)
        fire(c0 + 1, 1)
        drain_gather(1)
        out_start(c0 + 1, 1)
        drain_out(0)
        fire(c0 + 2, 0)
        return carry

    lax.fori_loop(1, _NPAIR - 1, body, 0)

    drain_gather(0)
    out_start(_NCHUNK - 2, 0)
    drain_out(1)
    fire(_NCHUNK - 1, 1)
    drain_gather(1)
    out_start(_NCHUNK - 1, 1)
    drain_out(0)
    drain_out(1)


def kernel(src_input, word_lut):
    idx = src_input.reshape(_SEQ, _BATCH)
    return _emb_lookup(idx, word_lut)
